# baseline (speedup 1.0000x reference)
import functools

import jax
import jax.numpy as jnp
from jax import lax
from jax.experimental import pallas as pl
from jax.experimental.pallas import tpu as pltpu
from jax.experimental.pallas import tpu_sc as plsc

VOCAB = 1000000
EMB = 32
B = 16384
L = 20
N = B * L
TBL_R = VOCAB * EMB // 128

NUM_CORES = 2
NUM_SUBCORES = 16
NW = NUM_CORES * NUM_SUBCORES
ROWS_PER_W = N // NW
CHUNK = 256
N_CHUNKS = ROWS_PER_W // CHUNK
NGRP = CHUNK // 16


def _make_gather():
    mesh = plsc.VectorSubcoreMesh(core_axis_name="c", subcore_axis_name="s")

    @functools.partial(
        pl.kernel,
        mesh=mesh,
        out_type=jax.ShapeDtypeStruct((N, EMB), jnp.float32),
        scratch_types=[
            pltpu.VMEM((ROWS_PER_W,), jnp.int32),
            pltpu.VMEM((CHUNK,), jnp.int32),
            pltpu.VMEM((CHUNK, 128), jnp.float32),
            pltpu.VMEM((CHUNK, EMB), jnp.float32),
            pltpu.SemaphoreType.DMA,
        ],
        compiler_params=pltpu.CompilerParams(needs_layout_passes=False),
    )
    def gather_kernel(idx_hbm, table_hbm, out_hbm, idx_v, q_v, rows_v, cb_v, sem):
        wid = lax.axis_index("s") * NUM_CORES + lax.axis_index("c")
        base = wid * ROWS_PER_W
        pltpu.sync_copy(idx_hbm.at[pl.ds(base, ROWS_PER_W)], idx_v)

        lanes = lax.iota(jnp.int32, 16)

        @pl.loop(0, N_CHUNKS)
        def _chunk(j):
            off = j * CHUNK
            for v in range(NGRP):
                iv = idx_v[pl.ds(off + v * 16, 16)]
                q_v[pl.ds(v * 16, 16)] = lax.shift_right_logical(iv, 2)
            pltpu.async_copy(table_hbm.at[q_v], rows_v, sem).wait()
            for v in range(NGRP):
                iv = idx_v[pl.ds(off + v * 16, 16)]
                cbase = lax.shift_left(jnp.bitwise_and(iv, 3), 5)
                rvec = lanes + (v * 16)
                for c in range(EMB):
                    x = plsc.load_gather(rows_v, [rvec, cbase + c])
                    plsc.store_scatter(
                        cb_v, [rvec, jnp.full((16,), c, jnp.int32)], x
                    )
            pltpu.sync_copy(cb_v, out_hbm.at[pl.ds(base + off, CHUNK)])

    return gather_kernel


_gather = _make_gather()


@jax.jit
def kernel(indices, table):
    flat_idx = indices.reshape(N)
    tbl128 = table.reshape(TBL_R, 128)
    out = _gather(flat_idx, tbl128)
    return out.reshape(B, L, EMB)


# speedup vs baseline: 1.7358x; 1.7358x over previous
import functools

import jax
import jax.numpy as jnp
from jax import lax
from jax.experimental import pallas as pl
from jax.experimental.pallas import tpu as pltpu
from jax.experimental.pallas import tpu_sc as plsc

VOCAB = 1000000
EMB = 32
B = 16384
L = 20
N = B * L

NUM_CORES = 2
NUM_SUBCORES = 16
NW = NUM_CORES * NUM_SUBCORES
B_PER_W = B // NW
ROWS_PER_W = B_PER_W * L
CHUNK_B = 64
CHUNK = CHUNK_B * L
N_CHUNKS = B_PER_W // CHUNK_B
NBUF = 2


def _make_gather():
    mesh = plsc.VectorSubcoreMesh(core_axis_name="c", subcore_axis_name="s")

    @functools.partial(
        pl.kernel,
        mesh=mesh,
        out_type=jax.ShapeDtypeStruct((B, L, EMB), jnp.float32),
        scratch_types=[
            pltpu.VMEM((ROWS_PER_W,), jnp.int32),
            pltpu.VMEM((NBUF, CHUNK, EMB), jnp.float32),
            pltpu.SemaphoreType.DMA((NBUF,)),
            pltpu.SemaphoreType.DMA((NBUF,)),
        ],
        compiler_params=pltpu.CompilerParams(use_tc_tiling_on_sc=False),
    )
    def gather_kernel(idx_hbm, table_hbm, out_hbm, idx_v, rows_v, gsem, wsem):
        wid = lax.axis_index("s") * NUM_CORES + lax.axis_index("c")
        base = wid * ROWS_PER_W
        b_base = wid * B_PER_W
        pltpu.sync_copy(idx_hbm.at[pl.ds(base, ROWS_PER_W)], idx_v)

        def gather_args(j, b):
            return (
                table_hbm.at[idx_v.at[pl.ds(j * CHUNK, CHUNK)]],
                rows_v.at[b],
                gsem.at[b],
            )

        def wb_args(j, b, g):
            return (
                rows_v.at[b, pl.ds(g * L, L)],
                out_hbm.at[b_base + j * CHUNK_B + g],
                wsem.at[b],
            )

        for j in range(N_CHUNKS):
            b = j % NBUF
            if j >= NBUF:
                for g in range(CHUNK_B):
                    pltpu.make_async_copy(*wb_args(j - NBUF, b, g)).wait()
            pltpu.async_copy(*gather_args(j, b))
            if j >= 1:
                bp = (j - 1) % NBUF
                pltpu.make_async_copy(*gather_args(j - 1, bp)).wait()
                for g in range(CHUNK_B):
                    pltpu.async_copy(*wb_args(j - 1, bp, g))
        b_last = (N_CHUNKS - 1) % NBUF
        pltpu.make_async_copy(*gather_args(N_CHUNKS - 1, b_last)).wait()
        for g in range(CHUNK_B):
            pltpu.async_copy(*wb_args(N_CHUNKS - 1, b_last, g))
        for j in range(N_CHUNKS - NBUF + 1, N_CHUNKS):
            for g in range(CHUNK_B):
                pltpu.make_async_copy(*wb_args(j, j % NBUF, g)).wait()

    return gather_kernel


_gather = _make_gather()


@jax.jit
def kernel(indices, table):
    flat_idx = indices.reshape(N)
    out = _gather(flat_idx, table)
    return out
